# baseline (speedup 1.0000x reference)
import functools

import jax
import jax.numpy as jnp
from jax import lax
from jax.experimental import pallas as pl
from jax.experimental.pallas import tpu as pltpu
from jax.experimental.pallas import tpu_sc as plsc

_ITEMS = 1000
_USERS = 16384
_TOPK = 10
_NW = 32
_UPW = _USERS // _NW
_CHUNK = 16
_NCHUNK = _UPW // _CHUNK
_XPU = 2 * _ITEMS
_DPU = 256
_XW = _CHUNK * _XPU
_DW = _CHUNK * _DPU
_STEPS = 62
_UNROLL = 8

_GDN = lax.GatherDimensionNumbers(
    offset_dims=(), collapsed_slice_dims=(0,), start_index_map=(0,))


def _vperm(v, idx):
    return lax.gather(v, idx[:, None], _GDN, (1,),
                      mode=lax.GatherScatterMode.PROMISE_IN_BOUNDS)


def _tree_sum(v):
    iota = lax.iota(jnp.int32, 16)
    for d in (8, 4, 2, 1):
        v = v + _vperm(v, iota ^ d)
    return v


def _sc_body(x_hbm, d_hbm, out_hbm, xb, db, resb):
    wid = lax.axis_index("s") * 2 + lax.axis_index("c")
    iota = lax.iota(jnp.int32, 16)
    podd = (iota % 2) == 1
    lane0 = iota == 0
    lane1 = iota == 1
    sh = ((iota >> 1) & 3) * 8
    grp = iota >> 3
    neginf = jnp.float32(-jnp.inf)
    one16 = jnp.ones((16,), jnp.int32)
    zero16 = jnp.zeros((16,), jnp.int32)
    zf16 = jnp.zeros((16,), jnp.float32)

    def user_metric(base, dbase):
        def dup_step(k, acc):
            wv = db[pl.ds(dbase + 16 * k, 16)]
            return acc + ((wv * 0x01010101) >> 24)
        dupv = lax.fori_loop(0, _DPU // 16, dup_step, zero16, unroll=8)
        ndup = _tree_sum(dupv)[0]
        tv = xb[pl.ds(base + _XPU - 16, 16)]
        wtail = db[pl.ds(dbase + 240, 16)]
        d999 = (wtail[9] >> 24) & 1
        tsplat = _vperm(tv, jnp.full((16,), 15, jnp.int32))

        def heavy():
            def step(s, cnt):
                a = xb[pl.ds(base + 32 * s, 16)]
                b = xb[pl.ds(base + 32 * s + 16, 16)]
                w16 = db[pl.ds(dbase + 16 * (s // 4), 16)]
                g4 = (s % 4) * 4
                wa = _vperm(w16, g4 + grp)
                wb = _vperm(w16, g4 + 2 + grp)
                da_ = (wa >> sh) & 1
                db_ = (wb >> sh) & 1
                xma = jnp.where(podd, a, neginf)
                xmb = jnp.where(podd, b, neginf)
                ca = jnp.where((xma >= tsplat) & (da_ == 0), one16, zero16)
                cb = jnp.where((xmb >= tsplat) & (db_ == 0), one16, zero16)
                return cnt + ca + cb
            cntv = lax.fori_loop(0, _STEPS, step, zero16, unroll=_UNROLL)
            wt = _vperm(wtail, 8 + grp)
            dt = (wt >> sh) & 1
            xmt = jnp.where(podd, tv, neginf)
            ct = jnp.where((xmt >= tsplat) & (dt == 0), one16, zero16)
            count = _tree_sum(cntv + ct)[0]
            return jnp.where(count <= _TOPK, 1.0, 0.0).astype(jnp.float32)

        hit = lax.cond(d999 == 0, heavy, lambda: jnp.float32(0))
        w = jnp.where(ndup != _ITEMS - 1, 1.0, 0.0).astype(jnp.float32)
        return hit * w, w

    def chunk_body(c, acc):
        u0 = wid * _UPW + c * _CHUNK
        pltpu.sync_copy(x_hbm.at[pl.ds(u0 * _XPU, _XW)], xb)
        pltpu.sync_copy(d_hbm.at[pl.ds(u0 * _DPU, _DW)], db)
        for u in range(_CHUNK):
            hit, w = user_metric(u * _XPU, u * _DPU)
            acc = acc + jnp.where(lane0, hit, zf16) + jnp.where(lane1, w, zf16)
        return acc

    acc = lax.fori_loop(0, _NCHUNK, chunk_body, zf16)
    resb[...] = acc
    pltpu.sync_copy(resb, out_hbm.at[pl.ds(wid * 16, 16)])


def kernel(logits, dup_mask):
    xf = logits.reshape(-1)
    d8 = dup_mask.astype(jnp.uint8).reshape(-1, 4)
    dwords = lax.bitcast_convert_type(d8, jnp.int32)
    dpad = jnp.pad(dwords.reshape(_USERS, _ITEMS // 4),
                   ((0, 0), (0, _DPU - _ITEMS // 4))).reshape(-1)
    mesh = plsc.VectorSubcoreMesh(core_axis_name="c", subcore_axis_name="s")
    sc = functools.partial(
        pl.kernel,
        mesh=mesh,
        out_type=jax.ShapeDtypeStruct((_NW * 16,), jnp.float32),
        scratch_types=[
            pltpu.VMEM((_XW,), jnp.float32),
            pltpu.VMEM((_DW,), jnp.int32),
            pltpu.VMEM((16,), jnp.float32),
        ],
    )(_sc_body)
    out = sc(xf, dpad)
    hr_sum = jnp.sum(out[0::16])
    hr_count = jnp.sum(out[1::16])
    return (logits, hr_sum, hr_count)


# speedup vs baseline: 1.0018x; 1.0018x over previous
import functools

import jax
import jax.numpy as jnp
from jax import lax
from jax.experimental import pallas as pl
from jax.experimental.pallas import tpu as pltpu
from jax.experimental.pallas import tpu_sc as plsc

_ITEMS = 1000
_USERS = 16384
_TOPK = 10
_NW = 32
_UPW = _USERS // _NW
_CHUNK = 16
_NCHUNK = _UPW // _CHUNK
_XPU = 2 * _ITEMS
_DPU = _ITEMS // 4
_XW = _CHUNK * _XPU
_DW = _CHUNK * _DPU
_STEPS = 62
_UNROLL = 8

_GDN = lax.GatherDimensionNumbers(
    offset_dims=(), collapsed_slice_dims=(0,), start_index_map=(0,))


def _vperm(v, idx):
    return lax.gather(v, idx[:, None], _GDN, (1,),
                      mode=lax.GatherScatterMode.PROMISE_IN_BOUNDS)


def _tree_sum(v):
    iota = lax.iota(jnp.int32, 16)
    for d in (8, 4, 2, 1):
        v = v + _vperm(v, iota ^ d)
    return v


def _sc_body(x_hbm, d_hbm, out_hbm, xb, db, resb):
    wid = lax.axis_index("s") * 2 + lax.axis_index("c")
    iota = lax.iota(jnp.int32, 16)
    podd = (iota % 2) == 1
    lane0 = iota == 0
    lane1 = iota == 1
    sh = ((iota >> 1) & 3) * 8
    grp = iota >> 3
    neginf = jnp.float32(-jnp.inf)
    one16 = jnp.ones((16,), jnp.int32)
    zero16 = jnp.zeros((16,), jnp.int32)
    zf16 = jnp.zeros((16,), jnp.float32)

    def user_metric(base, dbase):
        def dup_step(k, acc):
            wv = db[pl.ds(dbase + 16 * k, 16)]
            return acc + ((wv * 0x01010101) >> 24)
        dupv = lax.fori_loop(0, 15, dup_step, zero16, unroll=8)
        wtail = db[pl.ds(dbase + 234, 16)]
        dupv = dupv + jnp.where(iota >= 6,
                                ((wtail * 0x01010101) >> 24), zero16)
        ndup = _tree_sum(dupv)[0]
        tv = xb[pl.ds(base + _XPU - 16, 16)]
        d999 = (wtail[15] >> 24) & 1
        tsplat = _vperm(tv, jnp.full((16,), 15, jnp.int32))

        def heavy():
            def step(s, cnt):
                a = xb[pl.ds(base + 32 * s, 16)]
                b = xb[pl.ds(base + 32 * s + 16, 16)]
                w16 = db[pl.ds(dbase + 16 * (s // 4), 16)]
                g4 = (s % 4) * 4
                wa = _vperm(w16, g4 + grp)
                wb = _vperm(w16, g4 + 2 + grp)
                da_ = (wa >> sh) & 1
                db_ = (wb >> sh) & 1
                xma = jnp.where(podd, a, neginf)
                xmb = jnp.where(podd, b, neginf)
                ca = jnp.where((xma >= tsplat) & (da_ == 0), one16, zero16)
                cb = jnp.where((xmb >= tsplat) & (db_ == 0), one16, zero16)
                return cnt + ca + cb
            cntv = lax.fori_loop(0, _STEPS, step, zero16, unroll=_UNROLL)
            wt = _vperm(wtail, 14 + grp)
            dt = (wt >> sh) & 1
            xmt = jnp.where(podd, tv, neginf)
            ct = jnp.where((xmt >= tsplat) & (dt == 0), one16, zero16)
            count = _tree_sum(cntv + ct)[0]
            return jnp.where(count <= _TOPK, 1.0, 0.0).astype(jnp.float32)

        hit = lax.cond(d999 == 0, heavy, lambda: jnp.float32(0))
        w = jnp.where(ndup != _ITEMS - 1, 1.0, 0.0).astype(jnp.float32)
        return hit * w, w

    def chunk_body(c, acc):
        u0 = wid * _UPW + c * _CHUNK
        pltpu.sync_copy(x_hbm.at[pl.ds(u0 * _XPU, _XW)], xb)
        pltpu.sync_copy(d_hbm.at[pl.ds(u0 * _DPU, _DW)], db.at[pl.ds(0, _DW)])
        for u in range(_CHUNK):
            hit, w = user_metric(u * _XPU, u * _DPU)
            acc = acc + jnp.where(lane0, hit, zf16) + jnp.where(lane1, w, zf16)
        return acc

    acc = lax.fori_loop(0, _NCHUNK, chunk_body, zf16)
    resb[...] = acc
    pltpu.sync_copy(resb, out_hbm.at[pl.ds(wid * 16, 16)])


def kernel(logits, dup_mask):
    xf = logits.reshape(-1)
    d8 = dup_mask.astype(jnp.uint8).reshape(-1, 4)
    dwords = lax.bitcast_convert_type(d8, jnp.int32)
    mesh = plsc.VectorSubcoreMesh(core_axis_name="c", subcore_axis_name="s")
    sc = functools.partial(
        pl.kernel,
        mesh=mesh,
        out_type=jax.ShapeDtypeStruct((_NW * 16,), jnp.float32),
        scratch_types=[
            pltpu.VMEM((_XW,), jnp.float32),
            pltpu.VMEM((_DW + 16,), jnp.int32),
            pltpu.VMEM((16,), jnp.float32),
        ],
    )(_sc_body)
    out = sc(xf, dwords)
    hr_sum = jnp.sum(out[0::16])
    hr_count = jnp.sum(out[1::16])
    return (logits, hr_sum, hr_count)


# speedup vs baseline: 1.0372x; 1.0353x over previous
import functools

import jax
import jax.numpy as jnp
from jax import lax
from jax.experimental import pallas as pl
from jax.experimental.pallas import tpu as pltpu
from jax.experimental.pallas import tpu_sc as plsc

_ITEMS = 1000
_USERS = 16384
_TOPK = 10
_NW = 32
_UPW = _USERS // _NW
_CHUNK = 16
_NCHUNK = _UPW // _CHUNK
_XPU = 2 * _ITEMS
_XW = _CHUNK * _XPU
_DW = _CHUNK * _ITEMS
_STEPS = 62
_UNROLL = 8

_GDN = lax.GatherDimensionNumbers(
    offset_dims=(), collapsed_slice_dims=(0,), start_index_map=(0,))


def _vperm(v, idx):
    return lax.gather(v, idx[:, None], _GDN, (1,),
                      mode=lax.GatherScatterMode.PROMISE_IN_BOUNDS)


def _tree_sum(v):
    iota = lax.iota(jnp.int32, 16)
    for d in (8, 4, 2, 1):
        v = v + _vperm(v, iota ^ d)
    return v


def _sc_body(x_hbm, d_hbm, out_hbm, xb, db, resb):
    wid = lax.axis_index("s") * 2 + lax.axis_index("c")
    iota = lax.iota(jnp.int32, 16)
    odd_lo = 2 * (iota % 8) + 1
    half = iota >> 3
    neginf = jnp.float32(-jnp.inf)
    one16 = jnp.ones((16,), jnp.int32)
    zero16 = jnp.zeros((16,), jnp.int32)
    zf16 = jnp.zeros((16,), jnp.float32)

    def user_metric(base, dbase):
        def dup_step(k, acc):
            return acc + db[pl.ds(dbase + 16 * k, 16)]
        dupv = lax.fori_loop(0, _STEPS, dup_step, zero16, unroll=_UNROLL)
        dtail = db[pl.ds(dbase + _ITEMS - 16, 16)]
        dupv = dupv + jnp.where(iota >= 8, dtail, zero16)
        ndup = _tree_sum(dupv)[0]
        tv = xb[pl.ds(base + _XPU - 16, 16)]
        d999 = dtail[15]
        tsplat = _vperm(tv, jnp.full((16,), 15, jnp.int32))

        def heavy():
            def step(s, cnt):
                a = xb[pl.ds(base + 32 * s, 16)]
                b = xb[pl.ds(base + 32 * s + 16, 16)]
                dw = db[pl.ds(dbase + 16 * s, 16)]
                xo = jnp.where(half == 0, _vperm(a, odd_lo), _vperm(b, odd_lo))
                c = jnp.where((xo >= tsplat) & (dw == 0), one16, zero16)
                return cnt + c
            cntv = lax.fori_loop(0, _STEPS, step, zero16, unroll=_UNROLL)
            xt = _vperm(tv, odd_lo)
            dt = _vperm(dtail, 8 + (iota % 8))
            ct = jnp.where((iota < 8) & (xt >= tsplat) & (dt == 0),
                           one16, zero16)
            count = _tree_sum(cntv + ct)[0]
            return jnp.where(count <= _TOPK, 1.0, 0.0).astype(jnp.float32)

        hit = lax.cond(d999 == 0, heavy, lambda: jnp.float32(0))
        w = jnp.where(ndup != _ITEMS - 1, 1.0, 0.0).astype(jnp.float32)
        return hit * w, w

    def chunk_body(c, acc):
        u0 = wid * _UPW + c * _CHUNK
        pltpu.sync_copy(x_hbm.at[pl.ds(u0 * _XPU, _XW)], xb)
        pltpu.sync_copy(d_hbm.at[pl.ds(u0 * _ITEMS, _DW)], db)
        lane0 = iota == 0
        lane1 = iota == 1
        for u in range(_CHUNK):
            hit, w = user_metric(u * _XPU, u * _ITEMS)
            acc = acc + jnp.where(lane0, hit, zf16) + jnp.where(lane1, w, zf16)
        return acc

    acc = lax.fori_loop(0, _NCHUNK, chunk_body, zf16)
    resb[...] = acc
    pltpu.sync_copy(resb, out_hbm.at[pl.ds(wid * 16, 16)])


def kernel(logits, dup_mask):
    xf = logits.reshape(-1)
    dwords = dup_mask.reshape(-1).astype(jnp.int32)
    mesh = plsc.VectorSubcoreMesh(core_axis_name="c", subcore_axis_name="s")
    sc = functools.partial(
        pl.kernel,
        mesh=mesh,
        out_type=jax.ShapeDtypeStruct((_NW * 16,), jnp.float32),
        scratch_types=[
            pltpu.VMEM((_XW,), jnp.float32),
            pltpu.VMEM((_DW,), jnp.int32),
            pltpu.VMEM((16,), jnp.float32),
        ],
    )(_sc_body)
    out = sc(xf, dwords)
    hr_sum = jnp.sum(out[0::16])
    hr_count = jnp.sum(out[1::16])
    return (logits, hr_sum, hr_count)


# speedup vs baseline: 48.1190x; 46.3944x over previous
import functools

import jax
import jax.numpy as jnp
from jax import lax
from jax.experimental import pallas as pl
from jax.experimental.pallas import tpu as pltpu
from jax.experimental.pallas import tpu_sc as plsc

_ITEMS = 1000
_USERS = 16384
_TOPK = 10
_NW = 32
_UPW = _USERS // _NW
_CHUNK = 16
_NCHUNK = _UPW // _CHUNK
_CW = _CHUNK * _ITEMS
_STEPS = 62
_UNROLL = 8

_GDN = lax.GatherDimensionNumbers(
    offset_dims=(), collapsed_slice_dims=(0,), start_index_map=(0,))


def _vperm(v, idx):
    return lax.gather(v, idx[:, None], _GDN, (1,),
                      mode=lax.GatherScatterMode.PROMISE_IN_BOUNDS)


def _tree_sum(v):
    iota = lax.iota(jnp.int32, 16)
    for d in (8, 4, 2, 1):
        v = v + _vperm(v, iota ^ d)
    return v


def _sc_body(x_hbm, d_hbm, out_hbm, xb, db, resb):
    wid = lax.axis_index("s") * 2 + lax.axis_index("c")
    iota = lax.iota(jnp.int32, 16)
    one16 = jnp.ones((16,), jnp.int32)
    zero16 = jnp.zeros((16,), jnp.int32)
    zf16 = jnp.zeros((16,), jnp.float32)

    def user_metric(base):
        def dup_step(k, acc):
            return acc + db[pl.ds(base + 16 * k, 16)]
        dupv = lax.fori_loop(0, _STEPS, dup_step, zero16, unroll=_UNROLL)
        dtail = db[pl.ds(base + _ITEMS - 16, 16)]
        dupv = dupv + jnp.where(iota >= 8, dtail, zero16)
        ndup = _tree_sum(dupv)[0]
        tv = xb[pl.ds(base + _ITEMS - 16, 16)]
        d999 = dtail[15]
        tsplat = _vperm(tv, jnp.full((16,), 15, jnp.int32))

        def heavy():
            def step(s, cnt):
                xo = xb[pl.ds(base + 16 * s, 16)]
                dw = db[pl.ds(base + 16 * s, 16)]
                c = jnp.where((xo >= tsplat) & (dw == 0), one16, zero16)
                return cnt + c
            cntv = lax.fori_loop(0, _STEPS, step, zero16, unroll=_UNROLL)
            ct = jnp.where((iota >= 8) & (tv >= tsplat) & (dtail == 0),
                           one16, zero16)
            count = _tree_sum(cntv + ct)[0]
            return jnp.where(count <= _TOPK, 1.0, 0.0).astype(jnp.float32)

        hit = lax.cond(d999 == 0, heavy, lambda: jnp.float32(0))
        w = jnp.where(ndup != _ITEMS - 1, 1.0, 0.0).astype(jnp.float32)
        return hit * w, w

    def chunk_body(c, acc):
        r0 = (wid * _UPW + c * _CHUNK) * _ITEMS
        pltpu.sync_copy(x_hbm.at[pl.ds(r0, _CW)], xb)
        pltpu.sync_copy(d_hbm.at[pl.ds(r0, _CW)], db)
        lane0 = iota == 0
        lane1 = iota == 1
        for u in range(_CHUNK):
            hit, w = user_metric(u * _ITEMS)
            acc = acc + jnp.where(lane0, hit, zf16) + jnp.where(lane1, w, zf16)
        return acc

    acc = lax.fori_loop(0, _NCHUNK, chunk_body, zf16)
    resb[...] = acc
    pltpu.sync_copy(resb, out_hbm.at[pl.ds(wid * 16, 16)])


def kernel(logits, dup_mask):
    xcol = logits[:, 1]
    dwords = dup_mask.reshape(-1).astype(jnp.int32)
    mesh = plsc.VectorSubcoreMesh(core_axis_name="c", subcore_axis_name="s")
    sc = functools.partial(
        pl.kernel,
        mesh=mesh,
        out_type=jax.ShapeDtypeStruct((_NW * 16,), jnp.float32),
        scratch_types=[
            pltpu.VMEM((_CW,), jnp.float32),
            pltpu.VMEM((_CW,), jnp.int32),
            pltpu.VMEM((16,), jnp.float32),
        ],
    )(_sc_body)
    out = sc(xcol, dwords)
    hr_sum = jnp.sum(out[0::16])
    hr_count = jnp.sum(out[1::16])
    return (logits, hr_sum, hr_count)


# speedup vs baseline: 48.9529x; 1.0173x over previous
import functools

import jax
import jax.numpy as jnp
from jax import lax
from jax.experimental import pallas as pl
from jax.experimental.pallas import tpu as pltpu
from jax.experimental.pallas import tpu_sc as plsc

_ITEMS = 1000
_USERS = 16384
_TOPK = 10
_NW = 32
_UPW = _USERS // _NW
_CHUNK = 32
_NCHUNK = _UPW // _CHUNK
_CW = _CHUNK * _ITEMS
_STEPS = 62
_UNROLL = 8

_GDN = lax.GatherDimensionNumbers(
    offset_dims=(), collapsed_slice_dims=(0,), start_index_map=(0,))


def _vperm(v, idx):
    return lax.gather(v, idx[:, None], _GDN, (1,),
                      mode=lax.GatherScatterMode.PROMISE_IN_BOUNDS)


def _tree_sum(v):
    iota = lax.iota(jnp.int32, 16)
    for d in (8, 4, 2, 1):
        v = v + _vperm(v, iota ^ d)
    return v


def _sc_body(x_hbm, d_hbm, out_hbm, xb, db, resb, semx, semd):
    wid = lax.axis_index("s") * 2 + lax.axis_index("c")
    iota = lax.iota(jnp.int32, 16)
    one16 = jnp.ones((16,), jnp.int32)
    zero16 = jnp.zeros((16,), jnp.int32)
    zf16 = jnp.zeros((16,), jnp.float32)

    def user_metric(base):
        def dup_step(k, acc):
            return acc + db[pl.ds(base + 16 * k, 16)]
        dupv = lax.fori_loop(0, _STEPS, dup_step, zero16, unroll=_UNROLL)
        dtail = db[pl.ds(base + _ITEMS - 16, 16)]
        dupv = dupv + jnp.where(iota >= 8, dtail, zero16)
        ndup = _tree_sum(dupv)[0]
        tv = xb[pl.ds(base + _ITEMS - 16, 16)]
        d999 = dtail[15]
        tsplat = _vperm(tv, jnp.full((16,), 15, jnp.int32))

        def heavy():
            def step(s, cnt):
                xo = xb[pl.ds(base + 16 * s, 16)]
                dw = db[pl.ds(base + 16 * s, 16)]
                c = jnp.where((xo >= tsplat) & (dw == 0), one16, zero16)
                return cnt + c
            cntv = lax.fori_loop(0, _STEPS, step, zero16, unroll=_UNROLL)
            ct = jnp.where((iota >= 8) & (tv >= tsplat) & (dtail == 0),
                           one16, zero16)
            count = _tree_sum(cntv + ct)[0]
            return jnp.where(count <= _TOPK, 1.0, 0.0).astype(jnp.float32)

        hit = lax.cond(d999 == 0, heavy, lambda: jnp.float32(0))
        w = jnp.where(ndup != _ITEMS - 1, 1.0, 0.0).astype(jnp.float32)
        return hit * w, w

    def chunk_body(c, acc):
        r0 = (wid * _UPW + c * _CHUNK) * _ITEMS
        cx = pltpu.async_copy(x_hbm.at[pl.ds(r0, _CW)], xb, semx)
        cd = pltpu.async_copy(d_hbm.at[pl.ds(r0, _CW)], db, semd)
        cx.wait()
        cd.wait()
        lane0 = iota == 0
        lane1 = iota == 1
        for u in range(_CHUNK):
            hit, w = user_metric(u * _ITEMS)
            acc = acc + jnp.where(lane0, hit, zf16) + jnp.where(lane1, w, zf16)
        return acc

    acc = lax.fori_loop(0, _NCHUNK, chunk_body, zf16)
    resb[...] = acc
    pltpu.sync_copy(resb, out_hbm.at[pl.ds(wid * 16, 16)])


def kernel(logits, dup_mask):
    xcol = logits[:, 1]
    dwords = dup_mask.reshape(-1).astype(jnp.int32)
    mesh = plsc.VectorSubcoreMesh(core_axis_name="c", subcore_axis_name="s")
    sc = functools.partial(
        pl.kernel,
        mesh=mesh,
        out_type=jax.ShapeDtypeStruct((_NW * 16,), jnp.float32),
        scratch_types=[
            pltpu.VMEM((_CW,), jnp.float32),
            pltpu.VMEM((_CW,), jnp.int32),
            pltpu.VMEM((16,), jnp.float32),
            pltpu.SemaphoreType.DMA,
            pltpu.SemaphoreType.DMA,
        ],
    )(_sc_body)
    out = sc(xcol, dwords)
    hr_sum = jnp.sum(out[0::16])
    hr_count = jnp.sum(out[1::16])
    return (logits, hr_sum, hr_count)
